# baseline (speedup 1.0000x reference)
import functools

import jax
import jax.numpy as jnp
from jax import lax
from jax.experimental import pallas as pl
from jax.experimental.pallas import tpu as pltpu
from jax.experimental.pallas import tpu_sc as plsc

N_NODES = 10000
D = 128
N_GRAPHS = 64

_NC = 2
_NS = 16
_NW = _NC * _NS
_CHUNK = 128
_N_PAD = 10240
_ROWS_PER_TILE = _N_PAD // _NS
_ZROWS = 128


def _sc_weighted_segment_sum(y, sd4, ea3, nch):
    mesh = plsc.VectorSubcoreMesh(core_axis_name="c", subcore_axis_name="s")

    @functools.partial(
        pl.kernel,
        out_type=jax.ShapeDtypeStruct((_NC, _N_PAD, D), jnp.float32),
        mesh=mesh,
        scratch_types=[
            pltpu.VMEM((2, _CHUNK), jnp.int32),
            pltpu.VMEM((2, _CHUNK), jnp.int32),
            pltpu.VMEM((2, _CHUNK), jnp.int32),
            pltpu.VMEM((2, _CHUNK), jnp.int32),
            pltpu.VMEM((nch * _CHUNK,), jnp.float32),
            pltpu.VMEM((_CHUNK, D), jnp.float32),
            pltpu.VMEM((_CHUNK, D), jnp.float32),
            pltpu.VMEM_SHARED((_N_PAD, D), jnp.float32),
            pltpu.SemaphoreType.DMA,
            pltpu.SemaphoreType.DMA,
            pltpu.SemaphoreType.DMA,
            pltpu.SemaphoreType.DMA,
            pltpu.SemaphoreType.DMA,
            pltpu.SemaphoreType.DMA,
            pltpu.SemaphoreType.DMA,
            pltpu.SemaphoreType.DMA,
        ],
    )
    def agg_kernel(y_hbm, sd_hbm, ea_hbm, out_hbm,
                   sda0, sdb0, sda1, sdb1, ea_v, rows_a, rows_b, acc,
                   sem_ia0, sem_ib0, sem_ia1, sem_ib1,
                   sem_ga, sem_gb, sem_sa, sem_sb):
        c = lax.axis_index("c")
        s = lax.axis_index("s")
        wid = s * _NC + c

        pltpu.sync_copy(ea_hbm.at[wid], ea_v)

        def _zero_row(r, carry):
            for q in range(D // 16):
                rows_a[r, pl.ds(q * 16, 16)] = jnp.zeros((16,), jnp.float32)
            return carry
        lax.fori_loop(0, _ZROWS, _zero_row, 0)
        row0 = s * _ROWS_PER_TILE
        for k in range(_ROWS_PER_TILE // _ZROWS):
            pltpu.async_copy(rows_a, acc.at[pl.ds(row0 + k * _ZROWS, _ZROWS)],
                             sem_sa)
        for k in range(_ROWS_PER_TILE // _ZROWS):
            pltpu.make_async_copy(
                rows_a, acc.at[pl.ds(row0 + k * _ZROWS, _ZROWS)], sem_sa).wait()
        plsc.subcore_barrier()

        dnums = lax.GatherDimensionNumbers(
            offset_dims=(), collapsed_slice_dims=(0,), start_index_map=(0,))

        def _scale(rows_v, j):
            def _group(g, c2):
                wv = ea_v[pl.ds(j * _CHUNK + g * 16, 16)]
                r0 = g * 16
                for l in range(16):
                    w = lax.gather(
                        wv, jnp.full((16, 1), l, jnp.int32), dnums, (1,),
                        mode=lax.GatherScatterMode.PROMISE_IN_BOUNDS)
                    for q in range(D // 16):
                        rows_v[r0 + l, pl.ds(q * 16, 16)] = (
                            rows_v[r0 + l, pl.ds(q * 16, 16)] * w)
                return c2
            lax.fori_loop(0, _CHUNK // 16, _group, 0)

        def _wait_idx(buf, sem):
            pltpu.make_async_copy(sd_hbm.at[wid, 0], buf, sem).wait()

        def _wait_rows(rows_v, sem):
            pltpu.make_async_copy(y_hbm.at[sda0.at[0]], rows_v, sem).wait()

        def _wait_scat(rows_v, buf, sem):
            pltpu.make_async_copy(rows_v, acc.at[buf.at[1]], sem).wait()

        def _stage(ch, buf, sem):
            pltpu.async_copy(sd_hbm.at[wid, ch], buf, sem)

        def _gather(buf, rows_v, sem):
            pltpu.async_copy(y_hbm.at[buf.at[0]], rows_v, sem)

        def _scatter(rows_v, buf, sem):
            pltpu.async_copy(rows_v, acc.at[buf.at[1]], sem, add=True)

        nq = nch // 4

        pltpu.async_copy(sd_hbm.at[wid, 0], sda0, sem_ia0).wait()
        _gather(sda0, rows_a, sem_ga)
        _stage(1, sdb0, sem_ib0)
        _stage(2, sda1, sem_ia1)
        _stage(3, sdb1, sem_ib1)

        def _quad(q, carry):
            c = 4 * q
            @pl.when(q > 0)
            def _():
                _wait_scat(rows_b, sdb1, sem_sb)
                _stage(c + 3, sdb1, sem_ib1)
            _wait_idx(sdb0, sem_ib0)
            _gather(sdb0, rows_b, sem_gb)
            _wait_rows(rows_a, sem_ga)
            _scale(rows_a, c)
            _scatter(rows_a, sda0, sem_sa)
            _wait_rows(rows_b, sem_gb)
            _scale(rows_b, c + 1)
            _scatter(rows_b, sdb0, sem_sb)
            _wait_scat(rows_a, sda0, sem_sa)
            @pl.when(q < nq - 1)
            def _():
                _stage(c + 4, sda0, sem_ia0)
            _wait_idx(sda1, sem_ia1)
            _gather(sda1, rows_a, sem_ga)
            _wait_scat(rows_b, sdb0, sem_sb)
            @pl.when(q < nq - 1)
            def _():
                _stage(c + 5, sdb0, sem_ib0)
            _wait_idx(sdb1, sem_ib1)
            _gather(sdb1, rows_b, sem_gb)
            _wait_rows(rows_a, sem_ga)
            _scale(rows_a, c + 2)
            _scatter(rows_a, sda1, sem_sa)
            _wait_rows(rows_b, sem_gb)
            _scale(rows_b, c + 3)
            _scatter(rows_b, sdb1, sem_sb)
            _wait_scat(rows_a, sda1, sem_sa)
            @pl.when(q < nq - 1)
            def _():
                _stage(c + 6, sda1, sem_ia1)
                _wait_idx(sda0, sem_ia0)
                _gather(sda0, rows_a, sem_ga)
            return carry
        lax.fori_loop(0, nq, _quad, 0)

        _wait_scat(rows_b, sdb1, sem_sb)

        plsc.subcore_barrier()

        pltpu.sync_copy(
            acc.at[pl.ds(row0, _ROWS_PER_TILE)],
            out_hbm.at[c, pl.ds(row0, _ROWS_PER_TILE)],
        )

    return agg_kernel(y, sd4, ea3)


_BLK = 1000


def _mm2_body(x_ref, wa_ref, wb_ref, ya_ref, yb_ref):
    xb = x_ref[...]
    ya_ref[...] = jnp.dot(xb, wa_ref[...], preferred_element_type=jnp.float32, precision=lax.Precision.HIGHEST)
    yb_ref[...] = jnp.dot(xb, wb_ref[...], preferred_element_type=jnp.float32, precision=lax.Precision.HIGHEST)


def _tc_dual_matmul(x, wa_t, wb_t):
    return pl.pallas_call(
        _mm2_body,
        grid=(N_NODES // _BLK,),
        in_specs=[
            pl.BlockSpec((_BLK, D), lambda i: (i, 0)),
            pl.BlockSpec((D, D), lambda i: (0, 0)),
            pl.BlockSpec((D, D), lambda i: (0, 0)),
        ],
        out_specs=[pl.BlockSpec((_BLK, D), lambda i: (i, 0))] * 2,
        out_shape=[jax.ShapeDtypeStruct((N_NODES, D), jnp.float32)] * 2,
    )(x, wa_t, wb_t)


def _mid_body(agg_ref, r_ref, b_ref, wa_ref, wb_ref, ya_ref, yb_ref):
    h = agg_ref[0] + agg_ref[1] + b_ref[...] + r_ref[...]
    h = jnp.maximum(h, 0.0)
    ya_ref[...] = jnp.dot(h, wa_ref[...], preferred_element_type=jnp.float32, precision=lax.Precision.HIGHEST)
    yb_ref[...] = jnp.dot(h, wb_ref[...], preferred_element_type=jnp.float32, precision=lax.Precision.HIGHEST)


def _tc_mid(aggp, r1, b1, wa_t, wb_t):
    return pl.pallas_call(
        _mid_body,
        grid=(N_NODES // _BLK,),
        in_specs=[
            pl.BlockSpec((_NC, _BLK, D), lambda i: (0, i, 0)),
            pl.BlockSpec((_BLK, D), lambda i: (i, 0)),
            pl.BlockSpec((1, D), lambda i: (0, 0)),
            pl.BlockSpec((D, D), lambda i: (0, 0)),
            pl.BlockSpec((D, D), lambda i: (0, 0)),
        ],
        out_specs=[pl.BlockSpec((_BLK, D), lambda i: (i, 0))] * 2,
        out_shape=[jax.ShapeDtypeStruct((N_NODES, D), jnp.float32)] * 2,
    )(aggp, r1, b1, wa_t, wb_t)


def _head_body(agg_ref, r_ref, b_ref, batch_ref, wl_ref, bl_ref, out_ref):
    h2 = (agg_ref[0, :N_NODES, :] + agg_ref[1, :N_NODES, :]
          + b_ref[...] + r_ref[...])
    ids = lax.broadcasted_iota(jnp.int32, (N_GRAPHS, N_NODES), 0).astype(jnp.float32)
    mask = (batch_ref[...] == ids).astype(jnp.float32)
    sums = jnp.dot(mask, h2, preferred_element_type=jnp.float32, precision=lax.Precision.HIGHEST)
    counts = jnp.sum(mask, axis=1, keepdims=True)
    g = sums / jnp.maximum(counts, 1.0)
    o = jnp.sum(g * wl_ref[...], axis=1, keepdims=True) + bl_ref[0:1, 0:1]
    out_ref[...] = jnp.maximum(jnp.broadcast_to(o, (N_GRAPHS, D)), 0.0)


def _tc_head(aggp, r3, b3, batchf, w_lin, b_lin2):
    out128 = pl.pallas_call(
        _head_body,
        out_shape=jax.ShapeDtypeStruct((N_GRAPHS, D), jnp.float32),
    )(aggp, r3, b3, batchf, w_lin, b_lin2)
    return out128[:, :1]


def kernel(x, edge_index, batch, edge_attr,
           W_rel1, b_rel1, W_root1,
           W_rel3, b_rel3, W_root3,
           W_lin, b_lin):
    n_edges = edge_index.shape[1]
    grain = _NW * _CHUNK
    nch = -(-n_edges // grain)
    nch = ((nch + 3) // 4) * 4
    e_pad = nch * grain

    src = edge_index[0].astype(jnp.int32)
    dst = edge_index[1].astype(jnp.int32)
    ea = edge_attr.astype(jnp.float32)
    pad = e_pad - n_edges
    if pad:
        ar = jnp.arange(pad, dtype=jnp.int32)
        src = jnp.concatenate([src, ar % N_NODES])
        dst = jnp.concatenate([dst, N_NODES + ar % (_N_PAD - N_NODES)])
        ea = jnp.concatenate([ea, jnp.zeros((pad,), jnp.float32)])
    src3 = src.reshape(_NW, nch, 1, _CHUNK)
    dst3 = dst.reshape(_NW, nch, 1, _CHUNK)
    sd4 = jnp.concatenate([src3, dst3], axis=2)
    ea3 = ea.reshape(_NW, nch * _CHUNK)

    b1 = b_rel1.reshape(1, D)
    b3 = b_rel3.reshape(1, D)
    batchf = batch.astype(jnp.float32).reshape(1, N_NODES)
    b_lin2 = jnp.broadcast_to(b_lin.reshape(1, 1), (1, D))

    y1, r1 = _tc_dual_matmul(x, W_rel1.T, W_root1.T)
    aggp1 = _sc_weighted_segment_sum(y1, sd4, ea3, nch)

    y3, r3 = _tc_mid(aggp1, r1, b1, W_rel3.T, W_root3.T)
    aggp3 = _sc_weighted_segment_sum(y3, sd4, ea3, nch)

    return _tc_head(aggp3, r3, b3, batchf, W_lin, b_lin2)


# speedup vs baseline: 1.0051x; 1.0051x over previous
import functools

import jax
import jax.numpy as jnp
from jax import lax
from jax.experimental import pallas as pl
from jax.experimental.pallas import tpu as pltpu
from jax.experimental.pallas import tpu_sc as plsc

N_NODES = 10000
D = 128
N_GRAPHS = 64

_NC = 2
_NS = 16
_NW = _NC * _NS
_CHUNK = 128
_N_PAD = 10240
_ROWS_PER_TILE = _N_PAD // _NS
_ZROWS = 128


def _sc_weighted_segment_sum(y, sd4, ea3, nch):
    mesh = plsc.VectorSubcoreMesh(core_axis_name="c", subcore_axis_name="s")

    @functools.partial(
        pl.kernel,
        out_type=jax.ShapeDtypeStruct((_NC, _N_PAD, D), jnp.float32),
        mesh=mesh,
        scratch_types=[
            pltpu.VMEM((2, _CHUNK), jnp.int32),
            pltpu.VMEM((2, _CHUNK), jnp.int32),
            pltpu.VMEM((2, _CHUNK), jnp.int32),
            pltpu.VMEM((2, _CHUNK), jnp.int32),
            pltpu.VMEM((nch * _CHUNK,), jnp.float32),
            pltpu.VMEM((_CHUNK, D), jnp.float32),
            pltpu.VMEM((_CHUNK, D), jnp.float32),
            pltpu.VMEM_SHARED((_N_PAD, D), jnp.float32),
            pltpu.SemaphoreType.DMA,
            pltpu.SemaphoreType.DMA,
            pltpu.SemaphoreType.DMA,
            pltpu.SemaphoreType.DMA,
            pltpu.SemaphoreType.DMA,
            pltpu.SemaphoreType.DMA,
            pltpu.SemaphoreType.DMA,
            pltpu.SemaphoreType.DMA,
        ],
    )
    def agg_kernel(y_hbm, sd_hbm, ea_hbm, out_hbm,
                   sda0, sdb0, sda1, sdb1, ea_v, rows_a, rows_b, acc,
                   sem_ia0, sem_ib0, sem_ia1, sem_ib1,
                   sem_ga, sem_gb, sem_sa, sem_sb):
        c = lax.axis_index("c")
        s = lax.axis_index("s")
        wid = s * _NC + c

        pltpu.sync_copy(ea_hbm.at[wid], ea_v)

        def _zero_row(r, carry):
            for q in range(D // 16):
                rows_a[r, pl.ds(q * 16, 16)] = jnp.zeros((16,), jnp.float32)
            return carry
        lax.fori_loop(0, _ZROWS, _zero_row, 0)
        row0 = s * _ROWS_PER_TILE
        for k in range(_ROWS_PER_TILE // _ZROWS):
            pltpu.sync_copy(rows_a, acc.at[pl.ds(row0 + k * _ZROWS, _ZROWS)])
        plsc.subcore_barrier()

        dnums = lax.GatherDimensionNumbers(
            offset_dims=(), collapsed_slice_dims=(0,), start_index_map=(0,))

        def _scale(rows_v, j):
            def _group(g, c2):
                wv = ea_v[pl.ds(j * _CHUNK + g * 16, 16)]
                r0 = g * 16
                for l in range(16):
                    w = lax.gather(
                        wv, jnp.full((16, 1), l, jnp.int32), dnums, (1,),
                        mode=lax.GatherScatterMode.PROMISE_IN_BOUNDS)
                    for q in range(D // 16):
                        rows_v[r0 + l, pl.ds(q * 16, 16)] = (
                            rows_v[r0 + l, pl.ds(q * 16, 16)] * w)
                return c2
            lax.fori_loop(0, _CHUNK // 16, _group, 0)

        def _wait_idx(buf, sem):
            pltpu.make_async_copy(sd_hbm.at[wid, 0], buf, sem).wait()

        def _wait_rows(rows_v, sem):
            pltpu.make_async_copy(y_hbm.at[sda0.at[0]], rows_v, sem).wait()

        def _wait_scat(rows_v, buf, sem):
            pltpu.make_async_copy(rows_v, acc.at[buf.at[1]], sem).wait()

        def _stage(ch, buf, sem):
            pltpu.async_copy(sd_hbm.at[wid, ch], buf, sem)

        def _gather(buf, rows_v, sem):
            pltpu.async_copy(y_hbm.at[buf.at[0]], rows_v, sem)

        def _scatter(rows_v, buf, sem):
            pltpu.async_copy(rows_v, acc.at[buf.at[1]], sem, add=True)

        nq = nch // 4

        pltpu.async_copy(sd_hbm.at[wid, 0], sda0, sem_ia0).wait()
        _gather(sda0, rows_a, sem_ga)
        _stage(1, sdb0, sem_ib0)
        _stage(2, sda1, sem_ia1)
        _stage(3, sdb1, sem_ib1)

        def _quad(q, carry):
            c = 4 * q
            @pl.when(q > 0)
            def _():
                _wait_scat(rows_b, sdb1, sem_sb)
                _stage(c + 3, sdb1, sem_ib1)
            _wait_idx(sdb0, sem_ib0)
            _gather(sdb0, rows_b, sem_gb)
            _wait_rows(rows_a, sem_ga)
            _scale(rows_a, c)
            _scatter(rows_a, sda0, sem_sa)
            _wait_rows(rows_b, sem_gb)
            _scale(rows_b, c + 1)
            _scatter(rows_b, sdb0, sem_sb)
            _wait_scat(rows_a, sda0, sem_sa)
            @pl.when(q < nq - 1)
            def _():
                _stage(c + 4, sda0, sem_ia0)
            _wait_idx(sda1, sem_ia1)
            _gather(sda1, rows_a, sem_ga)
            _wait_scat(rows_b, sdb0, sem_sb)
            @pl.when(q < nq - 1)
            def _():
                _stage(c + 5, sdb0, sem_ib0)
            _wait_idx(sdb1, sem_ib1)
            _gather(sdb1, rows_b, sem_gb)
            _wait_rows(rows_a, sem_ga)
            _scale(rows_a, c + 2)
            _scatter(rows_a, sda1, sem_sa)
            _wait_rows(rows_b, sem_gb)
            _scale(rows_b, c + 3)
            _scatter(rows_b, sdb1, sem_sb)
            _wait_scat(rows_a, sda1, sem_sa)
            @pl.when(q < nq - 1)
            def _():
                _stage(c + 6, sda1, sem_ia1)
                _wait_idx(sda0, sem_ia0)
                _gather(sda0, rows_a, sem_ga)
            return carry
        lax.fori_loop(0, nq, _quad, 0)

        _wait_scat(rows_b, sdb1, sem_sb)

        plsc.subcore_barrier()

        for k in range(_ROWS_PER_TILE // _ZROWS):
            pltpu.sync_copy(
                acc.at[pl.ds(row0 + k * _ZROWS, _ZROWS)],
                out_hbm.at[c, pl.ds(row0 + k * _ZROWS, _ZROWS)],
            )

    return agg_kernel(y, sd4, ea3)


_BLK = 1000


def _mm2_body(x_ref, wa_ref, wb_ref, ya_ref, yb_ref):
    xb = x_ref[...]
    ya_ref[...] = jnp.dot(xb, wa_ref[...], preferred_element_type=jnp.float32, precision=lax.Precision.HIGHEST)
    yb_ref[...] = jnp.dot(xb, wb_ref[...], preferred_element_type=jnp.float32, precision=lax.Precision.HIGHEST)


def _tc_dual_matmul(x, wa_t, wb_t):
    return pl.pallas_call(
        _mm2_body,
        grid=(N_NODES // _BLK,),
        in_specs=[
            pl.BlockSpec((_BLK, D), lambda i: (i, 0)),
            pl.BlockSpec((D, D), lambda i: (0, 0)),
            pl.BlockSpec((D, D), lambda i: (0, 0)),
        ],
        out_specs=[pl.BlockSpec((_BLK, D), lambda i: (i, 0))] * 2,
        out_shape=[jax.ShapeDtypeStruct((N_NODES, D), jnp.float32)] * 2,
    )(x, wa_t, wb_t)


def _mid_body(agg_ref, r_ref, b_ref, wa_ref, wb_ref, ya_ref, yb_ref):
    h = agg_ref[0] + agg_ref[1] + b_ref[...] + r_ref[...]
    h = jnp.maximum(h, 0.0)
    ya_ref[...] = jnp.dot(h, wa_ref[...], preferred_element_type=jnp.float32, precision=lax.Precision.HIGHEST)
    yb_ref[...] = jnp.dot(h, wb_ref[...], preferred_element_type=jnp.float32, precision=lax.Precision.HIGHEST)


def _tc_mid(aggp, r1, b1, wa_t, wb_t):
    return pl.pallas_call(
        _mid_body,
        grid=(N_NODES // _BLK,),
        in_specs=[
            pl.BlockSpec((_NC, _BLK, D), lambda i: (0, i, 0)),
            pl.BlockSpec((_BLK, D), lambda i: (i, 0)),
            pl.BlockSpec((1, D), lambda i: (0, 0)),
            pl.BlockSpec((D, D), lambda i: (0, 0)),
            pl.BlockSpec((D, D), lambda i: (0, 0)),
        ],
        out_specs=[pl.BlockSpec((_BLK, D), lambda i: (i, 0))] * 2,
        out_shape=[jax.ShapeDtypeStruct((N_NODES, D), jnp.float32)] * 2,
    )(aggp, r1, b1, wa_t, wb_t)


def _head_body(agg_ref, r_ref, b_ref, batch_ref, wl_ref, bl_ref, out_ref):
    h2 = (agg_ref[0, :N_NODES, :] + agg_ref[1, :N_NODES, :]
          + b_ref[...] + r_ref[...])
    ids = lax.broadcasted_iota(jnp.int32, (N_GRAPHS, N_NODES), 0).astype(jnp.float32)
    mask = (batch_ref[...] == ids).astype(jnp.float32)
    sums = jnp.dot(mask, h2, preferred_element_type=jnp.float32, precision=lax.Precision.HIGHEST)
    counts = jnp.sum(mask, axis=1, keepdims=True)
    g = sums / jnp.maximum(counts, 1.0)
    o = jnp.sum(g * wl_ref[...], axis=1, keepdims=True) + bl_ref[0:1, 0:1]
    out_ref[...] = jnp.maximum(jnp.broadcast_to(o, (N_GRAPHS, D)), 0.0)


def _tc_head(aggp, r3, b3, batchf, w_lin, b_lin2):
    out128 = pl.pallas_call(
        _head_body,
        out_shape=jax.ShapeDtypeStruct((N_GRAPHS, D), jnp.float32),
    )(aggp, r3, b3, batchf, w_lin, b_lin2)
    return out128[:, :1]


def kernel(x, edge_index, batch, edge_attr,
           W_rel1, b_rel1, W_root1,
           W_rel3, b_rel3, W_root3,
           W_lin, b_lin):
    n_edges = edge_index.shape[1]
    grain = _NW * _CHUNK
    nch = -(-n_edges // grain)
    nch = ((nch + 3) // 4) * 4
    e_pad = nch * grain

    src = edge_index[0].astype(jnp.int32)
    dst = edge_index[1].astype(jnp.int32)
    ea = edge_attr.astype(jnp.float32)
    pad = e_pad - n_edges
    if pad:
        ar = jnp.arange(pad, dtype=jnp.int32)
        src = jnp.concatenate([src, ar % N_NODES])
        dst = jnp.concatenate([dst, N_NODES + ar % (_N_PAD - N_NODES)])
        ea = jnp.concatenate([ea, jnp.zeros((pad,), jnp.float32)])
    src3 = src.reshape(_NW, nch, 1, _CHUNK)
    dst3 = dst.reshape(_NW, nch, 1, _CHUNK)
    sd4 = jnp.concatenate([src3, dst3], axis=2)
    ea3 = ea.reshape(_NW, nch * _CHUNK)

    b1 = b_rel1.reshape(1, D)
    b3 = b_rel3.reshape(1, D)
    batchf = batch.astype(jnp.float32).reshape(1, N_NODES)
    b_lin2 = jnp.broadcast_to(b_lin.reshape(1, 1), (1, D))

    y1, r1 = _tc_dual_matmul(x, W_rel1.T, W_root1.T)
    aggp1 = _sc_weighted_segment_sum(y1, sd4, ea3, nch)

    y3, r3 = _tc_mid(aggp1, r1, b1, W_rel3.T, W_root3.T)
    aggp3 = _sc_weighted_segment_sum(y3, sd4, ea3, nch)

    return _tc_head(aggp3, r3, b3, batchf, W_lin, b_lin2)
